# baseline (speedup 1.0000x reference)
import jax
import jax.numpy as jnp
import numpy as np
from jax.experimental import pallas as pl
from jax.experimental.pallas import tpu as pltpu

C = 2048
A = 24
N = C * A
HID = 128
LAT = 256
RBF = 16
CUT = 6.0
MAXZ = 100
LAYERS = 2

CB = 8
BA = CB * A
NB = C // CB

_SIG2 = (CUT / RBF) ** 2
_INV2S = 1.0 / (2.0 * _SIG2)
_CENTERS = np.linspace(0.0, CUT, RBF)


def _block_kernel(z_ref, frac_ref, types_ref, len_ref, ang_ref,
                  emb_ref, Wz_ref, bz_ref, wrbf_ref, W1_ref, b1_ref,
                  wf_ref, Watom_ref, batom_ref, F_ref, logit_ref):
    f32 = jnp.float32

    ang = ang_ref[:] * (np.pi / 180.0)
    cosang = jnp.cos(ang)
    ca, cb_, cg = cosang[:, 0], cosang[:, 1], cosang[:, 2]
    sg = jnp.clip(jnp.sin(ang[:, 2]), 1e-6, None)
    ln = len_ref[:]
    a, b, c = ln[:, 0], ln[:, 1], ln[:, 2]
    cy = (ca - cb_ * cg) / sg
    cz = jnp.sqrt(jnp.clip(1.0 - cb_ ** 2 - cy ** 2, 1e-6, None))
    l00 = a
    l10 = b * cg
    l11 = b * sg
    l20 = c * cb_
    l21 = c * cy
    l22 = c * cz

    def prow(x):
        return jnp.broadcast_to(x[:, None, None], (CB, A, 1)).reshape(BA, 1)

    frac = frac_ref[:]
    fx, fy, fz = frac[:, 0], frac[:, 1], frac[:, 2]
    dx = fx[:, None] - fx[None, :]
    dx = dx - jnp.round(dx)
    dy = fy[:, None] - fy[None, :]
    dy = dy - jnp.round(dy)
    dz = fz[:, None] - fz[None, :]
    dz = dz - jnp.round(dz)
    cxx = dx * prow(l00) + dy * prow(l10) + dz * prow(l20)
    cyy = dy * prow(l11) + dz * prow(l21)
    czz = dz * prow(l22)
    d = jnp.sqrt(cxx * cxx + cyy * cyy + czz * czz + 1e-8)

    pid = jax.lax.broadcasted_iota(jnp.int32, (BA, BA), 0)
    qid = jax.lax.broadcasted_iota(jnp.int32, (BA, BA), 1)
    keep = ((pid // A) == (qid // A)) & (pid != qid)
    maskf = jnp.where(keep, f32(1.0), f32(0.0))
    env = jnp.maximum(1.0 - d * (1.0 / CUT), 0.0)
    env = env * env * maskf

    w0 = jnp.zeros((BA, BA), f32)
    w1 = jnp.zeros((BA, BA), f32)
    for r in range(RBF):
        e = jnp.exp((d - _CENTERS[r]) ** 2 * (-_INV2S))
        w0 = w0 + e * wrbf_ref[0, r]
        w1 = w1 + e * wrbf_ref[1, r]

    t = jnp.clip(types_ref[0, 0, :] - 1, 0, MAXZ - 1)
    oh = (t[:, None] == jax.lax.broadcasted_iota(jnp.int32, (BA, MAXZ), 1)
          ).astype(f32)
    Hemb = jnp.dot(oh, emb_ref[:], preferred_element_type=f32, precision=jax.lax.Precision.HIGHEST)
    Hz = jnp.dot(z_ref[:], Wz_ref[:], preferred_element_type=f32, precision=jax.lax.Precision.HIGHEST) + bz_ref[:][None, :]
    H = Hemb + jnp.broadcast_to(Hz[:, None, :], (CB, A, HID)).reshape(BA, HID)

    for l in range(LAYERS):
        wl = (w0 if l == 0 else w1) * env
        m = jnp.dot(wl, H, preferred_element_type=f32, precision=jax.lax.Precision.HIGHEST)
        H = H + jax.nn.relu(
            jnp.dot(m, W1_ref[l], preferred_element_type=f32, precision=jax.lax.Precision.HIGHEST) + b1_ref[l][None, :])

    Hw = H * wf_ref[:][None, :]
    s = jax.lax.dot_general(Hw, H, (((1,), (1,)), ((), ())),
                            preferred_element_type=f32, precision=jax.lax.Precision.HIGHEST)
    s = s * env
    inv_d = 1.0 / d
    Fx = jnp.sum(s * (cxx * inv_d), axis=1)
    Fy = jnp.sum(s * (cyy * inv_d), axis=1)
    Fz = jnp.sum(s * (czz * inv_d), axis=1)
    F_ref[:] = jnp.stack([Fx, Fy, Fz], axis=-1)

    logit_ref[:] = (jnp.dot(H, Watom_ref[:], preferred_element_type=f32, precision=jax.lax.Precision.HIGHEST)
                    + batom_ref[:][None, :])


def kernel(z, pred_frac_coords, pred_atom_types, num_atoms, lengths, angles,
           atom_emb, Wz, bz, w_rbf, W1, b1, w_f, W_atom, b_atom):
    del num_atoms
    types3 = pred_atom_types.reshape(NB, 1, BA)

    def rep(shape):
        return pl.BlockSpec(shape, lambda i: (0,) * len(shape))

    F, logits = pl.pallas_call(
        _block_kernel,
        grid=(NB,),
        in_specs=[
            pl.BlockSpec((CB, LAT), lambda i: (i, 0)),
            pl.BlockSpec((BA, 3), lambda i: (i, 0)),
            pl.BlockSpec((1, 1, BA), lambda i: (i, 0, 0)),
            pl.BlockSpec((CB, 3), lambda i: (i, 0)),
            pl.BlockSpec((CB, 3), lambda i: (i, 0)),
            rep((MAXZ, HID)),
            rep((LAT, HID)),
            rep((HID,)),
            rep((LAYERS, RBF)),
            rep((LAYERS, HID, HID)),
            rep((LAYERS, HID)),
            rep((HID,)),
            rep((HID, MAXZ)),
            rep((MAXZ,)),
        ],
        out_specs=(pl.BlockSpec((BA, 3), lambda i: (i, 0)),
                   pl.BlockSpec((BA, MAXZ), lambda i: (i, 0))),
        out_shape=(jax.ShapeDtypeStruct((N, 3), jnp.float32),
                   jax.ShapeDtypeStruct((N, MAXZ), jnp.float32)),
        compiler_params=pltpu.CompilerParams(
            dimension_semantics=("parallel",)),
    )(z, pred_frac_coords, types3, lengths, angles, atom_emb, Wz, bz,
      w_rbf, W1, b1, w_f, W_atom, b_atom)
    return (F, logits)


# speedup vs baseline: 1.7460x; 1.7460x over previous
import jax
import jax.numpy as jnp
import numpy as np
from jax.experimental import pallas as pl
from jax.experimental.pallas import tpu as pltpu

C = 2048
A = 24
N = C * A
HID = 128
LAT = 256
RBF = 16
CUT = 6.0
MAXZ = 100
LAYERS = 2

CB = 8
BA = CB * A
NB = C // CB

_SIG2 = (CUT / RBF) ** 2
_INV2S = 1.0 / (2.0 * _SIG2)
_DELTA = CUT / (RBF - 1)
_UK = _DELTA / _SIG2
_KR = np.exp(-(2.0 * np.arange(RBF - 1) + 1.0) * _DELTA ** 2 * _INV2S)

_cid = np.arange(BA) // A
_MASK = ((_cid[:, None] == _cid[None, :])
         & (np.arange(BA)[:, None] != np.arange(BA)[None, :])
         ).astype(np.float32)


def _block_kernel(z_ref, frac_ref, types_ref, len_ref, ang_ref, mask_ref,
                  emb_ref, Wz_ref, bz_ref, wrbf_ref, W1_ref, b1_ref,
                  wf_ref, Watom_ref, batom_ref, F_ref, logit_ref):
    f32 = jnp.float32

    ang = ang_ref[:] * (np.pi / 180.0)
    cosang = jnp.cos(ang)
    ca, cb_, cg = cosang[:, 0], cosang[:, 1], cosang[:, 2]
    sg = jnp.clip(jnp.sin(ang[:, 2]), 1e-6, None)
    ln = len_ref[:]
    a, b, c = ln[:, 0], ln[:, 1], ln[:, 2]
    cy = (ca - cb_ * cg) / sg
    cz = jnp.sqrt(jnp.clip(1.0 - cb_ ** 2 - cy ** 2, 1e-6, None))
    l00 = a
    l10 = b * cg
    l11 = b * sg
    l20 = c * cb_
    l21 = c * cy
    l22 = c * cz

    def prow(x):
        return jnp.broadcast_to(x[:, None, None], (CB, A, 1)).reshape(BA, 1)

    frac = frac_ref[:]
    fx, fy, fz = frac[:, 0], frac[:, 1], frac[:, 2]
    dx = fx[:, None] - fx[None, :]
    dx = dx - jnp.round(dx)
    dy = fy[:, None] - fy[None, :]
    dy = dy - jnp.round(dy)
    dz = fz[:, None] - fz[None, :]
    dz = dz - jnp.round(dz)
    cxx = dx * prow(l00) + dy * prow(l10) + dz * prow(l20)
    cyy = dy * prow(l11) + dz * prow(l21)
    czz = dz * prow(l22)
    d2 = cxx * cxx + cyy * cyy + czz * czz + 1e-8
    inv_d = jax.lax.rsqrt(d2)
    d = d2 * inv_d
    dc = jnp.minimum(d, CUT)

    maskf = mask_ref[:]
    env = 1.0 - dc * (1.0 / CUT)
    env = env * env * maskf

    e = jnp.exp(dc * dc * (-_INV2S))
    u = jnp.exp(dc * _UK)
    w0 = e * wrbf_ref[0, 0]
    w1 = e * wrbf_ref[1, 0]
    for r in range(RBF - 1):
        e = (e * u) * _KR[r]
        w0 = w0 + e * wrbf_ref[0, r + 1]
        w1 = w1 + e * wrbf_ref[1, r + 1]

    t = jnp.clip(types_ref[0, 0, :] - 1, 0, MAXZ - 1)
    oh = (t[:, None] == jax.lax.broadcasted_iota(jnp.int32, (BA, MAXZ), 1)
          ).astype(f32)
    Hemb = jnp.dot(oh, emb_ref[:], preferred_element_type=f32)
    Hz = jnp.dot(z_ref[:], Wz_ref[:], preferred_element_type=f32) + bz_ref[:][None, :]
    H = Hemb + jnp.broadcast_to(Hz[:, None, :], (CB, A, HID)).reshape(BA, HID)

    for l in range(LAYERS):
        wl = (w0 if l == 0 else w1) * env
        m = jnp.dot(wl, H, preferred_element_type=f32)
        H = H + jax.nn.relu(
            jnp.dot(m, W1_ref[l], preferred_element_type=f32) + b1_ref[l][None, :])

    Hw = H * wf_ref[:][None, :]
    s = jax.lax.dot_general(Hw, H, (((1,), (1,)), ((), ())),
                            preferred_element_type=f32)
    s = s * env
    Fx = jnp.sum(s * (cxx * inv_d), axis=1)
    Fy = jnp.sum(s * (cyy * inv_d), axis=1)
    Fz = jnp.sum(s * (czz * inv_d), axis=1)
    F_ref[:] = jnp.stack([Fx, Fy, Fz], axis=-1)

    logit_ref[:] = (jnp.dot(H, Watom_ref[:], preferred_element_type=f32)
                    + batom_ref[:][None, :])


def kernel(z, pred_frac_coords, pred_atom_types, num_atoms, lengths, angles,
           atom_emb, Wz, bz, w_rbf, W1, b1, w_f, W_atom, b_atom):
    del num_atoms
    types3 = pred_atom_types.reshape(NB, 1, BA)
    mask = jnp.asarray(_MASK)

    def rep(shape):
        return pl.BlockSpec(shape, lambda i: (0,) * len(shape))

    F, logits = pl.pallas_call(
        _block_kernel,
        grid=(NB,),
        in_specs=[
            pl.BlockSpec((CB, LAT), lambda i: (i, 0)),
            pl.BlockSpec((BA, 3), lambda i: (i, 0)),
            pl.BlockSpec((1, 1, BA), lambda i: (i, 0, 0)),
            pl.BlockSpec((CB, 3), lambda i: (i, 0)),
            pl.BlockSpec((CB, 3), lambda i: (i, 0)),
            rep((BA, BA)),
            rep((MAXZ, HID)),
            rep((LAT, HID)),
            rep((HID,)),
            rep((LAYERS, RBF)),
            rep((LAYERS, HID, HID)),
            rep((LAYERS, HID)),
            rep((HID,)),
            rep((HID, MAXZ)),
            rep((MAXZ,)),
        ],
        out_specs=(pl.BlockSpec((BA, 3), lambda i: (i, 0)),
                   pl.BlockSpec((BA, MAXZ), lambda i: (i, 0))),
        out_shape=(jax.ShapeDtypeStruct((N, 3), jnp.float32),
                   jax.ShapeDtypeStruct((N, MAXZ), jnp.float32)),
        compiler_params=pltpu.CompilerParams(
            dimension_semantics=("parallel",)),
    )(z, pred_frac_coords, types3, lengths, angles, mask, atom_emb, Wz, bz,
      w_rbf, W1, b1, w_f, W_atom, b_atom)
    return (F, logits)
